# baseline (speedup 1.0000x reference)
import functools

import jax
import jax.numpy as jnp
from jax import lax
from jax.experimental import pallas as pl
from jax.experimental.pallas import tpu as pltpu
from jax.experimental.pallas import tpu_sc as plsc

BATCH = 16384
EMBED_DIM = 64
NUM_TABLES = 3
PAIR_DIM = 2 * EMBED_DIM
NC = 2
NS = 16
NW = NC * NS
B_PER_W = BATCH // NW
CHUNK = 128
N_CHUNKS = B_PER_W // CHUNK
LANES = 16

_MESH = plsc.VectorSubcoreMesh(core_axis_name="c", subcore_axis_name="s")


@functools.partial(
    pl.kernel,
    out_type=[jax.ShapeDtypeStruct((BATCH, PAIR_DIM), jnp.float32)
              for _ in range(NUM_TABLES)],
    mesh=_MESH,
    scratch_types=[
        [pltpu.VMEM((B_PER_W,), jnp.int32) for _ in range(NUM_TABLES)],
        [pltpu.VMEM((B_PER_W,), jnp.int32) for _ in range(NUM_TABLES)],
        [pltpu.VMEM((CHUNK, PAIR_DIM), jnp.float32) for _ in range(6)],
        pltpu.SemaphoreType.DMA,
        pltpu.SemaphoreType.DMA,
    ],
)
def _pair_gather(iu_hbm, if_hbm, ia_hbm, w_user2, w_fans2, w_avg2,
                 out_u, out_f, out_a, idxs, pairs, bufs, sem_g, sem_w):
    wid = lax.axis_index("s") * NC + lax.axis_index("c")
    base = wid * B_PER_W
    for src, idx_v in zip((iu_hbm, if_hbm, ia_hbm), idxs):
        pltpu.sync_copy(src.at[pl.ds(base, B_PER_W)], idx_v)
    for idx_v, pair_v in zip(idxs, pairs):
        for k in range(B_PER_W // LANES):
            sl = pl.ds(k * LANES, LANES)
            pair_v[sl] = lax.shift_right_logical(idx_v[sl], 1)

    tables = (w_user2, w_fans2, w_avg2)
    outs = (out_u, out_f, out_a)
    gather_d = [None, None]
    write_d = [None, None]

    def fire_gathers(c, b):
        gather_d[b] = [
            pltpu.async_copy(
                tables[t].at[pairs[t].at[pl.ds(c * CHUNK, CHUNK)]],
                bufs[3 * b + t], sem_g)
            for t in range(NUM_TABLES)]

    def fire_writes(c, b):
        write_d[b] = [
            pltpu.async_copy(
                bufs[3 * b + t],
                outs[t].at[pl.ds(base + c * CHUNK, CHUNK), :], sem_w)
            for t in range(NUM_TABLES)]

    fire_gathers(0, 0)
    for c in range(N_CHUNKS):
        b = c % 2
        if c + 1 < N_CHUNKS:
            nb = (c + 1) % 2
            if write_d[nb] is not None:
                for d in write_d[nb]:
                    d.wait()
                write_d[nb] = None
            fire_gathers(c + 1, nb)
        for d in gather_d[b]:
            d.wait()
        fire_writes(c, b)
    for ds_ in write_d:
        if ds_ is not None:
            for d in ds_:
                d.wait()


_RS_BLK = 2000


def _merge_body(x_ref, o_ref):
    x3 = x_ref[...].reshape(_RS_BLK // 2, 2, EMBED_DIM)
    o_ref[...] = jnp.concatenate((x3[:, 0, :], x3[:, 1, :]), axis=1)


_pair_merge = pl.pallas_call(
    _merge_body,
    grid=(100000 // _RS_BLK,),
    in_specs=[pl.BlockSpec((_RS_BLK, EMBED_DIM), lambda i: (i, 0))],
    out_specs=pl.BlockSpec((_RS_BLK // 2, PAIR_DIM), lambda i: (i, 0)),
    out_shape=jax.ShapeDtypeStruct((50000, PAIR_DIM), jnp.float32),
)


_ROWS_PER_BLK = 1024


def _select_body(pu_ref, pf_ref, pa_ref, par_ref, o_ref):
    for t, blk_ref in enumerate((pu_ref, pf_ref, pa_ref)):
        q2 = par_ref[:, pl.ds(t, 1)] == 1
        blk = blk_ref[...]
        sel = jnp.where(q2, blk[:, EMBED_DIM:PAIR_DIM], blk[:, 0:EMBED_DIM])
        o_ref[:, t * EMBED_DIM:(t + 1) * EMBED_DIM] = sel


_select_concat = pl.pallas_call(
    _select_body,
    grid=(BATCH // _ROWS_PER_BLK,),
    in_specs=[pl.BlockSpec((_ROWS_PER_BLK, PAIR_DIM), lambda i: (i, 0))
              for _ in range(NUM_TABLES)]
    + [pl.BlockSpec((_ROWS_PER_BLK, 8), lambda i: (i, 0))],
    out_specs=pl.BlockSpec((_ROWS_PER_BLK, NUM_TABLES * EMBED_DIM),
                           lambda i: (i, 0)),
    out_shape=jax.ShapeDtypeStruct((BATCH, NUM_TABLES * EMBED_DIM),
                                   jnp.float32),
)


def kernel(user_fea, W_user, W_fans, W_avg):
    idx = user_fea.T
    pair_tables = [_pair_merge(w) for w in (W_user, W_fans, W_avg)]
    pu, pf, pa = _pair_gather(idx[0], idx[1], idx[2], *pair_tables)
    par = jnp.pad(user_fea & 1, ((0, 0), (0, 8 - NUM_TABLES)))
    return _select_concat(pu, pf, pa, par)


# speedup vs baseline: 1.6390x; 1.6390x over previous
import functools

import jax
import jax.numpy as jnp
from jax import lax
from jax.experimental import pallas as pl
from jax.experimental.pallas import tpu as pltpu
from jax.experimental.pallas import tpu_sc as plsc

BATCH = 16384
EMBED_DIM = 64
NUM_TABLES = 3
NC = 2
NS = 16
NW = NC * NS
B_PER_W = BATCH // NW
CHUNK = 128
N_CHUNKS = B_PER_W // CHUNK

_MESH = plsc.VectorSubcoreMesh(core_axis_name="c", subcore_axis_name="s")


@functools.partial(
    pl.kernel,
    out_type=jax.ShapeDtypeStruct((BATCH, NUM_TABLES * EMBED_DIM), jnp.float32),
    mesh=_MESH,
    scratch_types=[
        pltpu.VMEM((B_PER_W,), jnp.int32),
        pltpu.VMEM((B_PER_W,), jnp.int32),
        pltpu.VMEM((B_PER_W,), jnp.int32),
        pltpu.VMEM((B_PER_W, EMBED_DIM), jnp.float32),
        pltpu.VMEM((B_PER_W, EMBED_DIM), jnp.float32),
        pltpu.VMEM((B_PER_W, EMBED_DIM), jnp.float32),
        pltpu.SemaphoreType.DMA,
    ],
    compiler_params=pltpu.CompilerParams(use_tc_tiling_on_sc=False),
)
def _emb_kernel(iu_hbm, if_hbm, ia_hbm, w_user, w_fans, w_avg, out_hbm,
                idx_u, idx_f, idx_a, rows_u, rows_f, rows_a, sem):
    wid = lax.axis_index("s") * NC + lax.axis_index("c")
    base = wid * B_PER_W
    for src, idx_v in ((iu_hbm, idx_u), (if_hbm, idx_f), (ia_hbm, idx_a)):
        pltpu.sync_copy(src.at[pl.ds(base, B_PER_W)], idx_v)
    copies = []
    for idx_v, table, rows in ((idx_u, w_user, rows_u),
                               (idx_f, w_fans, rows_f),
                               (idx_a, w_avg, rows_a)):
        for c in range(N_CHUNKS):
            copies.append(pltpu.async_copy(
                table.at[idx_v.at[pl.ds(c * CHUNK, CHUNK)]],
                rows.at[pl.ds(c * CHUNK, CHUNK), :],
                sem))
    for cp in copies:
        cp.wait()
    for t, rows in enumerate((rows_u, rows_f, rows_a)):
        pltpu.sync_copy(
            rows,
            out_hbm.at[pl.ds(base, B_PER_W),
                       pl.ds(t * EMBED_DIM, EMBED_DIM)])


def kernel(user_fea, W_user, W_fans, W_avg):
    idx = user_fea.T
    return _emb_kernel(idx[0], idx[1], idx[2], W_user, W_fans, W_avg)
